# baseline (speedup 1.0000x reference)
import jax
import jax.numpy as jnp
from jax import lax
from jax.experimental import pallas as pl
from jax.experimental.pallas import tpu as pltpu

B = 1024
T = 200
D = 128
_H = B // 2
_UNROLL = 8


def _eos_gather_body(x_hbm, text_ref, out_ref, flat_v, flat_s, sem0, sem1):
    def count_half(h):
        t = text_ref[pl.ds(h * _H, _H), :]
        nz = (t != 0).astype(jnp.float32)
        ones = jnp.ones((T, 1), jnp.float32)
        cnt = lax.dot_general(
            nz, ones, (((1,), (0,)), ((), ())),
            preferred_element_type=jnp.float32,
        ).reshape(_H).astype(jnp.int32)
        eos = jnp.clip(cnt - 1, 0, T - 1)
        base = (lax.broadcasted_iota(jnp.int32, (_H,), 0) + h * _H) * T
        flat_v[pl.ds(h * _H, _H)] = base + eos

    def smem_copy(h):
        return pltpu.make_async_copy(
            flat_v.at[pl.ds(h * _H, _H)], flat_s.at[pl.ds(h * _H, _H)], sem0
        )

    def issue_half(h):
        def issue(i, carry):
            for u in range(_UNROLL):
                ii = h * _H + i * _UNROLL + u
                r = flat_s[ii]
                pltpu.make_async_copy(
                    x_hbm.at[pl.ds(r, 1)], out_ref.at[pl.ds(ii, 1)], sem1
                ).start(priority=u % 2)
            return carry

        lax.fori_loop(0, _H // _UNROLL, issue, 0)

    count_half(0)
    smem_copy(0).start()
    count_half(1)
    smem_copy(1).start()
    smem_copy(0).wait()
    issue_half(0)
    smem_copy(1).wait()
    issue_half(1)

    pltpu.make_async_copy(x_hbm.at[pl.ds(0, B)], out_ref, sem1).wait()


@jax.jit
def kernel(x, text):
    x2 = x.reshape(B * T, D)
    text32 = text.astype(jnp.int32)
    return pl.pallas_call(
        _eos_gather_body,
        in_specs=[
            pl.BlockSpec(memory_space=pl.ANY),
            pl.BlockSpec(memory_space=pltpu.VMEM),
        ],
        out_specs=pl.BlockSpec(memory_space=pltpu.VMEM),
        out_shape=jax.ShapeDtypeStruct((B, D), jnp.float32),
        scratch_shapes=[
            pltpu.VMEM((B,), jnp.int32),
            pltpu.SMEM((B,), jnp.int32),
            pltpu.SemaphoreType.DMA,
            pltpu.SemaphoreType.DMA,
        ],
    )(x2, text32)


# speedup vs baseline: 1.0278x; 1.0278x over previous
import jax
import jax.numpy as jnp
from jax import lax
from jax.experimental import pallas as pl
from jax.experimental.pallas import tpu as pltpu

B = 1024
T = 200
D = 128
_H = B // 2
_Q = B // 4
_UNROLL = 8


def _eos_gather_body(x_hbm, text_ref, out_hbm, flat_v, flat_s, out_v,
                     sem0, sg0, sg1, sg2, sg3, sem_o):
    sg = (sg0, sg1, sg2, sg3)

    def count_half(h):
        t = text_ref[pl.ds(h * _H, _H), :]
        cnt = jnp.sum((t != 0).astype(jnp.int32), axis=1)
        eos = jnp.clip(cnt - 1, 0, T - 1)
        base = (lax.broadcasted_iota(jnp.int32, (_H,), 0) + h * _H) * T
        flat_v[pl.ds(h * _H, _H)] = base + eos

    def smem_copy(h):
        return pltpu.make_async_copy(
            flat_v.at[pl.ds(h * _H, _H)], flat_s.at[pl.ds(h * _H, _H)], sem0
        )

    def issue_quarter(q):
        def issue(i, carry):
            for u in range(_UNROLL):
                ii = q * _Q + i * _UNROLL + u
                r = flat_s[ii]
                pltpu.make_async_copy(
                    x_hbm.at[pl.ds(r, 1)], out_v.at[pl.ds(ii, 1)], sg[q]
                ).start(priority=u % 2)
            return carry

        lax.fori_loop(0, _Q // _UNROLL, issue, 0)

    def out_copy(q):
        return pltpu.make_async_copy(
            out_v.at[pl.ds(q * _Q, _Q)], out_hbm.at[pl.ds(q * _Q, _Q)], sem_o
        )

    count_half(0)
    smem_copy(0).start()
    count_half(1)
    smem_copy(1).start()
    smem_copy(0).wait()
    issue_quarter(0)
    issue_quarter(1)
    smem_copy(1).wait()
    issue_quarter(2)
    issue_quarter(3)

    for q in range(4):
        pltpu.make_async_copy(
            x_hbm.at[pl.ds(0, _Q)], out_v.at[pl.ds(q * _Q, _Q)], sg[q]
        ).wait()
        out_copy(q).start(priority=q % 2)
    for q in range(4):
        out_copy(q).wait()


@jax.jit
def kernel(x, text):
    x2 = x.reshape(B * T, D)
    text32 = text.astype(jnp.int32)
    return pl.pallas_call(
        _eos_gather_body,
        in_specs=[
            pl.BlockSpec(memory_space=pl.ANY),
            pl.BlockSpec(memory_space=pltpu.VMEM),
        ],
        out_specs=pl.BlockSpec(memory_space=pl.ANY),
        out_shape=jax.ShapeDtypeStruct((B, D), jnp.float32),
        scratch_shapes=[
            pltpu.VMEM((B,), jnp.int32),
            pltpu.SMEM((B,), jnp.int32),
            pltpu.VMEM((B, D), jnp.float32),
        ] + [pltpu.SemaphoreType.DMA] * 6,
    )(x2, text32)


# speedup vs baseline: 1.0306x; 1.0027x over previous
import jax
import jax.numpy as jnp
from jax import lax
from jax.experimental import pallas as pl
from jax.experimental.pallas import tpu as pltpu

B = 1024
T = 200
D = 128
_H = B // 2
_Q = B // 4
_UNROLL = 8


def _eos_gather_body(x_hbm, text_ref, out_hbm, flat_v, flat_s, out_v,
                     sem0, sg0, sg1, sg2, sg3, sem_o):
    sg = (sg0, sg1, sg2, sg3)

    def count_rows(lo, n):
        t = text_ref[pl.ds(lo, n), :]
        cnt = jnp.sum((t != 0).astype(jnp.int32), axis=1)
        eos = jnp.clip(cnt - 1, 0, T - 1)
        base = (lax.broadcasted_iota(jnp.int32, (n,), 0) + lo) * T
        flat_v[pl.ds(lo, n)] = base + eos

    def smem_copy(lo, n):
        return pltpu.make_async_copy(
            flat_v.at[pl.ds(lo, n)], flat_s.at[pl.ds(lo, n)], sem0
        )

    def issue_quarter(q):
        def issue(i, carry):
            for u in range(_UNROLL):
                ii = q * _Q + i * _UNROLL + u
                r = flat_s[ii]
                pltpu.make_async_copy(
                    x_hbm.at[pl.ds(r, 1)], out_v.at[pl.ds(ii, 1)], sg[q]
                ).start(priority=u % 2)
            return carry

        lax.fori_loop(0, _Q // _UNROLL, issue, 0)

    def out_copy(q):
        return pltpu.make_async_copy(
            out_v.at[pl.ds(q * _Q, _Q)], out_hbm.at[pl.ds(q * _Q, _Q)], sem_o
        )

    count_rows(0, _Q)
    smem_copy(0, _Q).start()
    count_rows(_Q, 3 * _Q)
    smem_copy(_Q, 3 * _Q).start()
    smem_copy(0, _Q).wait()
    issue_quarter(0)
    smem_copy(_Q, 3 * _Q).wait()
    issue_quarter(1)
    issue_quarter(2)
    issue_quarter(3)

    for q in range(4):
        pltpu.make_async_copy(
            x_hbm.at[pl.ds(0, _Q)], out_v.at[pl.ds(q * _Q, _Q)], sg[q]
        ).wait()
        out_copy(q).start(priority=q % 2)
    for q in range(4):
        out_copy(q).wait()


@jax.jit
def kernel(x, text):
    x2 = x.reshape(B * T, D)
    text32 = text.astype(jnp.int32)
    return pl.pallas_call(
        _eos_gather_body,
        in_specs=[
            pl.BlockSpec(memory_space=pl.ANY),
            pl.BlockSpec(memory_space=pltpu.VMEM),
        ],
        out_specs=pl.BlockSpec(memory_space=pl.ANY),
        out_shape=jax.ShapeDtypeStruct((B, D), jnp.float32),
        scratch_shapes=[
            pltpu.VMEM((B,), jnp.int32),
            pltpu.SMEM((B,), jnp.int32),
            pltpu.VMEM((B, D), jnp.float32),
        ] + [pltpu.SemaphoreType.DMA] * 6,
    )(x2, text32)


# speedup vs baseline: 1.0308x; 1.0002x over previous
import jax
import jax.numpy as jnp
from jax import lax
from jax.experimental import pallas as pl
from jax.experimental.pallas import tpu as pltpu

B = 1024
T = 200
D = 128
_H = B // 2
_Q = B // 4
_UNROLL = 8


def _eos_gather_body(x_hbm, text_ref, out_hbm, flat_v, flat_s, out_v,
                     sem0, sem0b, sg0, sg1, sg2, sg3, sem_o):
    sg = (sg0, sg1, sg2, sg3)

    def count_rows(lo, n):
        t = text_ref[pl.ds(lo, n), :]
        cnt = jnp.sum((t != 0).astype(jnp.int32), axis=1)
        eos = jnp.clip(cnt - 1, 0, T - 1)
        base = (lax.broadcasted_iota(jnp.int32, (n,), 0) + lo) * T
        flat_v[pl.ds(lo, n)] = base + eos

    def smem_copy(lo, n, sem):
        return pltpu.make_async_copy(
            flat_v.at[pl.ds(lo, n)], flat_s.at[pl.ds(lo, n)], sem
        )

    def issue_quarter(q):
        def issue(i, carry):
            for u in range(_UNROLL):
                ii = q * _Q + i * _UNROLL + u
                r = flat_s[ii]
                pltpu.make_async_copy(
                    x_hbm.at[pl.ds(r, 1)], out_v.at[pl.ds(ii, 1)], sg[q]
                ).start(priority=u % 2)
            return carry

        lax.fori_loop(0, _Q // _UNROLL, issue, 0)

    def out_copy(q):
        return pltpu.make_async_copy(
            out_v.at[pl.ds(q * _Q, _Q)], out_hbm.at[pl.ds(q * _Q, _Q)], sem_o
        )

    count_rows(0, _Q)
    smem_copy(0, _Q, sem0).start()
    count_rows(_Q, 3 * _Q)
    smem_copy(_Q, 3 * _Q, sem0b).start()
    smem_copy(0, _Q, sem0).wait()
    issue_quarter(0)
    smem_copy(_Q, 3 * _Q, sem0b).wait()
    issue_quarter(1)
    issue_quarter(2)
    issue_quarter(3)

    for q in range(4):
        pltpu.make_async_copy(
            x_hbm.at[pl.ds(0, _Q)], out_v.at[pl.ds(q * _Q, _Q)], sg[q]
        ).wait()
        out_copy(q).start(priority=q % 2)
    for q in range(4):
        out_copy(q).wait()


@jax.jit
def kernel(x, text):
    x2 = x.reshape(B * T, D)
    text32 = text.astype(jnp.int32)
    return pl.pallas_call(
        _eos_gather_body,
        in_specs=[
            pl.BlockSpec(memory_space=pl.ANY),
            pl.BlockSpec(memory_space=pltpu.VMEM),
        ],
        out_specs=pl.BlockSpec(memory_space=pl.ANY),
        out_shape=jax.ShapeDtypeStruct((B, D), jnp.float32),
        scratch_shapes=[
            pltpu.VMEM((B,), jnp.int32),
            pltpu.SMEM((B,), jnp.int32),
            pltpu.VMEM((B, D), jnp.float32),
        ] + [pltpu.SemaphoreType.DMA] * 7,
    )(x2, text32)


# speedup vs baseline: 1.0560x; 1.0245x over previous
import jax
import jax.numpy as jnp
from jax import lax
from jax.experimental import pallas as pl
from jax.experimental.pallas import tpu as pltpu

B = 1024
T = 200
D = 128
_H = B // 2
_Q = B // 4
_UNROLL = 16


def _eos_gather_body(x_hbm, text_ref, out_hbm, flat_v, flat_s, out_v,
                     sem0, sem0b, sg0, sg1, sg2, sg3, sem_o):
    sg = (sg0, sg1, sg2, sg3)

    def count_rows(lo, n):
        t = text_ref[pl.ds(lo, n), :]
        cnt = jnp.sum((t != 0).astype(jnp.int32), axis=1)
        eos = jnp.clip(cnt - 1, 0, T - 1)
        base = (lax.broadcasted_iota(jnp.int32, (n,), 0) + lo) * T
        flat_v[pl.ds(lo, n)] = base + eos

    def smem_copy(lo, n, sem):
        return pltpu.make_async_copy(
            flat_v.at[pl.ds(lo, n)], flat_s.at[pl.ds(lo, n)], sem
        )

    def issue_quarter(q):
        def issue(i, carry):
            for u in range(_UNROLL):
                ii = q * _Q + i * _UNROLL + u
                r = flat_s[ii]
                pltpu.make_async_copy(
                    x_hbm.at[pl.ds(r, 1)], out_v.at[pl.ds(ii, 1)], sg[q]
                ).start(priority=u % 2)
            return carry

        lax.fori_loop(0, _Q // _UNROLL, issue, 0)

    def out_copy(q):
        return pltpu.make_async_copy(
            out_v.at[pl.ds(q * _Q, _Q)], out_hbm.at[pl.ds(q * _Q, _Q)], sem_o
        )

    count_rows(0, _Q)
    smem_copy(0, _Q, sem0).start()
    count_rows(_Q, 3 * _Q)
    smem_copy(_Q, 3 * _Q, sem0b).start()
    smem_copy(0, _Q, sem0).wait()
    issue_quarter(0)
    smem_copy(_Q, 3 * _Q, sem0b).wait()
    issue_quarter(1)
    issue_quarter(2)
    issue_quarter(3)

    for q in range(4):
        pltpu.make_async_copy(
            x_hbm.at[pl.ds(0, _Q)], out_v.at[pl.ds(q * _Q, _Q)], sg[q]
        ).wait()
        out_copy(q).start(priority=q % 2)
    for q in range(4):
        out_copy(q).wait()


@jax.jit
def kernel(x, text):
    x2 = x.reshape(B * T, D)
    text32 = text.astype(jnp.int32)
    return pl.pallas_call(
        _eos_gather_body,
        in_specs=[
            pl.BlockSpec(memory_space=pl.ANY),
            pl.BlockSpec(memory_space=pltpu.VMEM),
        ],
        out_specs=pl.BlockSpec(memory_space=pl.ANY),
        out_shape=jax.ShapeDtypeStruct((B, D), jnp.float32),
        scratch_shapes=[
            pltpu.VMEM((B,), jnp.int32),
            pltpu.SMEM((B,), jnp.int32),
            pltpu.VMEM((B, D), jnp.float32),
        ] + [pltpu.SemaphoreType.DMA] * 7,
    )(x2, text32)


# speedup vs baseline: 1.0727x; 1.0157x over previous
import jax
import jax.numpy as jnp
from jax import lax
from jax.experimental import pallas as pl
from jax.experimental.pallas import tpu as pltpu

B = 1024
T = 200
D = 128
_H = B // 2
_Q = B // 4
_UNROLL = 32


def _eos_gather_body(x_hbm, text_ref, out_hbm, flat_v, flat_s, out_v,
                     sem0, sem0b, sg0, sg1, sg2, sg3, sem_o):
    sg = (sg0, sg1, sg2, sg3)

    def count_rows(lo, n):
        t = text_ref[pl.ds(lo, n), :]
        cnt = jnp.sum((t != 0).astype(jnp.int32), axis=1)
        eos = jnp.clip(cnt - 1, 0, T - 1)
        base = (lax.broadcasted_iota(jnp.int32, (n,), 0) + lo) * T
        flat_v[pl.ds(lo, n)] = base + eos

    def smem_copy(lo, n, sem):
        return pltpu.make_async_copy(
            flat_v.at[pl.ds(lo, n)], flat_s.at[pl.ds(lo, n)], sem
        )

    def issue_quarter(q):
        def issue(i, carry):
            for u in range(_UNROLL):
                ii = q * _Q + i * _UNROLL + u
                r = flat_s[ii]
                pltpu.make_async_copy(
                    x_hbm.at[pl.ds(r, 1)], out_v.at[pl.ds(ii, 1)], sg[q]
                ).start(priority=u % 2)
            return carry

        lax.fori_loop(0, _Q // _UNROLL, issue, 0)

    def out_copy(q):
        return pltpu.make_async_copy(
            out_v.at[pl.ds(q * _Q, _Q)], out_hbm.at[pl.ds(q * _Q, _Q)], sem_o
        )

    count_rows(0, _Q)
    smem_copy(0, _Q, sem0).start()
    count_rows(_Q, 3 * _Q)
    smem_copy(_Q, 3 * _Q, sem0b).start()
    smem_copy(0, _Q, sem0).wait()
    issue_quarter(0)
    smem_copy(_Q, 3 * _Q, sem0b).wait()
    issue_quarter(1)
    issue_quarter(2)
    issue_quarter(3)

    for q in range(4):
        pltpu.make_async_copy(
            x_hbm.at[pl.ds(0, _Q)], out_v.at[pl.ds(q * _Q, _Q)], sg[q]
        ).wait()
        out_copy(q).start(priority=q % 2)
    for q in range(4):
        out_copy(q).wait()


@jax.jit
def kernel(x, text):
    x2 = x.reshape(B * T, D)
    text32 = text.astype(jnp.int32)
    return pl.pallas_call(
        _eos_gather_body,
        in_specs=[
            pl.BlockSpec(memory_space=pl.ANY),
            pl.BlockSpec(memory_space=pltpu.VMEM),
        ],
        out_specs=pl.BlockSpec(memory_space=pl.ANY),
        out_shape=jax.ShapeDtypeStruct((B, D), jnp.float32),
        scratch_shapes=[
            pltpu.VMEM((B,), jnp.int32),
            pltpu.SMEM((B,), jnp.int32),
            pltpu.VMEM((B, D), jnp.float32),
        ] + [pltpu.SemaphoreType.DMA] * 7,
    )(x2, text32)
